# baseline (device time: 197318 ns/iter reference)
import jax
import jax.numpy as jnp
from jax import lax
from jax.experimental import pallas as pl
from jax.experimental.pallas import tpu as pltpu

VB = 512


def kernel(x, W, labels):
    T, D = x.shape
    _, V = W.shape
    NB = V // VB

    xb = jnp.asarray(x, jnp.bfloat16)
    labels2d = labels.reshape(T, 1)

    def body(xb_ref, w_ref, lab_ref, out_ref,
             lbuf_ref, sacc_ref, stats_ref, rstats_ref, send_sem, recv_sem):
        j = pl.program_id(0)
        my_x = lax.axis_index("x")
        partner = (1 - my_x, lax.axis_index("y"), lax.axis_index("z"))

        @pl.when(j == 0)
        def _():
            sacc_ref[...] = jnp.zeros_like(sacc_ref)
            lbuf_ref[1] = jnp.full((T, VB), -1e4, jnp.bfloat16)
            barrier = pltpu.get_barrier_semaphore()
            pl.semaphore_signal(barrier, inc=1, device_id=partner,
                                device_id_type=pl.DeviceIdType.MESH)
            pl.semaphore_wait(barrier, 1)

        def update_stats(logits_bf16, blk_idx):
            logits = logits_bf16.astype(jnp.float32)
            rel = lab_ref[...] - (my_x * V + blk_idx * VB)
            cols = lax.broadcasted_iota(jnp.int32, (T, VB), 1)
            mask = cols == rel
            s_blk = jnp.sum(jnp.exp(logits), axis=1, keepdims=True)
            ll_blk = jnp.sum(
                jnp.where(mask, logits_bf16, jnp.bfloat16(0.0)),
                axis=1, keepdims=True, dtype=jnp.float32)
            sacc_ref[:, 0:1] = sacc_ref[:, 0:1] + s_blk
            sacc_ref[:, 1:2] = sacc_ref[:, 1:2] + ll_blk

        wb = w_ref[...].astype(jnp.bfloat16)
        logits = lax.dot_general(
            xb_ref[...], wb,
            dimension_numbers=(((1,), (0,)), ((), ())),
            preferred_element_type=jnp.float32)

        prev_idx = jnp.where(j == 0, jnp.int32(-(2 ** 20)), j - 1)
        update_stats(lbuf_ref[(j + 1) % 2], prev_idx)
        lbuf_ref[j % 2] = logits.astype(jnp.bfloat16)

        @pl.when(j == NB - 1)
        def _():
            update_stats(lbuf_ref[(NB - 1) % 2], NB - 1)
            stats_ref[0:1, :] = jnp.reshape(sacc_ref[:, 0:1], (1, T))
            stats_ref[1:2, :] = jnp.reshape(sacc_ref[:, 1:2], (1, T))
            rdma = pltpu.make_async_remote_copy(
                src_ref=stats_ref, dst_ref=rstats_ref,
                send_sem=send_sem, recv_sem=recv_sem,
                device_id=partner, device_id_type=pl.DeviceIdType.MESH)
            rdma.start()
            rdma.wait()
            s = stats_ref[0:1, :] + rstats_ref[0:1, :]
            ll = stats_ref[1:2, :] + rstats_ref[1:2, :]
            out_ref[...] = jnp.log(s) - ll

    out = pl.pallas_call(
        body,
        grid=(NB,),
        out_shape=jax.ShapeDtypeStruct((1, T), jnp.float32),
        in_specs=[
            pl.BlockSpec((T, D), lambda j: (0, 0)),
            pl.BlockSpec((D, VB), lambda j: (0, j)),
            pl.BlockSpec((T, 1), lambda j: (0, 0)),
        ],
        out_specs=pl.BlockSpec((1, T), lambda j: (0, 0)),
        scratch_shapes=[
            pltpu.VMEM((2, T, VB), jnp.bfloat16),
            pltpu.VMEM((T, 8), jnp.float32),
            pltpu.VMEM((8, T), jnp.float32),
            pltpu.VMEM((8, T), jnp.float32),
            pltpu.SemaphoreType.DMA,
            pltpu.SemaphoreType.DMA,
        ],
        compiler_params=pltpu.CompilerParams(
            collective_id=0,
            dimension_semantics=("arbitrary",),
            vmem_limit_bytes=60 * 1024 * 1024,
        ),
    )(xb, W, labels2d)
    return out.reshape(T)
